# baseline (speedup 1.0000x reference)
import functools

import jax
import jax.numpy as jnp
from jax import lax
from jax.experimental import pallas as pl
from jax.experimental.pallas import tpu as pltpu
from jax.experimental.pallas import tpu_sc as plsc

B, D, K = 1024, 1000, 16
DP = 1008
NC, NS, L = 2, 16, 16
NW = NC * NS
ROWS_PER_W = B // NW
GROUPS = ROWS_PER_W // L

_mesh = plsc.VectorSubcoreMesh(
    core_axis_name="c", subcore_axis_name="s", num_cores=NC, num_subcores=NS)


@functools.partial(
    pl.kernel,
    out_type=jax.ShapeDtypeStruct((B, DP), jnp.float32),
    mesh=_mesh,
    scratch_types=[
        pltpu.VMEM((L, D), jnp.float32),
        pltpu.VMEM((L, D), jnp.float32),
        pltpu.VMEM((L, DP), jnp.float32),
        pltpu.VMEM((L, DP), jnp.float32),
        pltpu.SemaphoreType.DMA,
        pltpu.SemaphoreType.DMA,
    ],
    compiler_params=pltpu.CompilerParams(needs_layout_passes=False),
)
def _sc_histogram(inputs_hbm, c_hbm, blk0, blk1, h0, h1, sem0, sem1):
    wid = lax.axis_index("s") * NC + lax.axis_index("c")
    lanes = lax.iota(jnp.int32, L)
    ones = jnp.ones((L,), jnp.float32)
    zeros16 = jnp.zeros((L,), jnp.float32)
    row0 = wid * ROWS_PER_W
    row1 = row0 + L

    cp0 = pltpu.async_copy(inputs_hbm.at[pl.ds(row0, L)], blk0, sem0)
    cp1 = pltpu.async_copy(inputs_hbm.at[pl.ds(row1, L)], blk1, sem1)

    @plsc.parallel_loop(0, DP // L, step=1)
    def _zero(i):
        off = i * L
        for l in range(L):
            h0[l, pl.ds(off, L)] = zeros16
            h1[l, pl.ds(off, L)] = zeros16

    cp0.wait()
    cp1.wait()

    @plsc.parallel_loop(0, D, step=1, unroll=8,
                        carry=lanes)
    def _scatter0(d, dvec):
        iv0 = plsc.load_gather(blk0, [lanes, dvec])
        plsc.addupdate_scatter(h0, [lanes, iv0.astype(jnp.int32)], ones)
        nd = dvec + 1
        return jnp.where(nd >= D, nd - D, nd)

    wb0 = pltpu.async_copy(h0, c_hbm.at[pl.ds(row0, L)], sem0)

    @plsc.parallel_loop(0, D, step=1, unroll=8,
                        carry=lanes)
    def _scatter1(d, dvec):
        iv1 = plsc.load_gather(blk1, [lanes, dvec])
        plsc.addupdate_scatter(h1, [lanes, iv1.astype(jnp.int32)], ones)
        nd = dvec + 1
        return jnp.where(nd >= D, nd - D, nd)

    wb1 = pltpu.async_copy(h1, c_hbm.at[pl.ds(row1, L)], sem1)
    wb0.wait()
    wb1.wait()


BLK = 512


def _tc_logits0(x_ref, w_ref, o_ref):
    o_ref[...] = jnp.dot(x_ref[...], w_ref[...],
                         preferred_element_type=jnp.float32)


def _tc_fm(c_ref, vp_ref, l0_ref, o_ref):
    vp = vp_ref[...]
    r = jnp.sum(vp * vp, axis=1, keepdims=True)
    c = c_ref[...]
    s = jnp.dot(c, vp, preferred_element_type=jnp.float32)
    t1 = jnp.dot(c, r, preferred_element_type=jnp.float32)
    res = l0_ref[...] + 0.5 * (
        jnp.sum(s * s, axis=1, keepdims=True) - t1)
    o_ref[...] = jnp.sum(res, axis=1)


@jax.jit
def kernel(inputs, w, v):
    l0 = pl.pallas_call(
        _tc_logits0,
        grid=(B // BLK,),
        in_specs=[
            pl.BlockSpec((BLK, D), lambda i: (i, 0)),
            pl.BlockSpec((D, 1), lambda i: (0, 0)),
        ],
        out_specs=pl.BlockSpec((BLK, 1), lambda i: (i, 0)),
        out_shape=jax.ShapeDtypeStruct((B, 1), jnp.float32),
    )(inputs, w)
    c = _sc_histogram(inputs)
    v_pad = jnp.pad(v, ((0, DP - D), (0, 0)))
    out = pl.pallas_call(
        _tc_fm,
        grid=(B // BLK,),
        in_specs=[
            pl.BlockSpec((BLK, DP), lambda i: (i, 0)),
            pl.BlockSpec((DP, K), lambda i: (0, 0)),
            pl.BlockSpec((BLK, 1), lambda i: (i, 0)),
        ],
        out_specs=pl.BlockSpec((BLK,), lambda i: (i,)),
        out_shape=jax.ShapeDtypeStruct((B,), jnp.float32),
    )(c, v_pad, l0)
    return out


# speedup vs baseline: 1.0407x; 1.0407x over previous
import functools

import jax
import jax.numpy as jnp
from jax import lax
from jax.experimental import pallas as pl
from jax.experimental.pallas import tpu as pltpu
from jax.experimental.pallas import tpu_sc as plsc

B, D, K = 1024, 1000, 16
DP = 1008
NC, NS, L = 2, 16, 16
NW = NC * NS
ROWS_PER_W = B // NW
GROUPS = ROWS_PER_W // L

_mesh = plsc.VectorSubcoreMesh(
    core_axis_name="c", subcore_axis_name="s", num_cores=NC, num_subcores=NS)


@functools.partial(
    pl.kernel,
    out_type=jax.ShapeDtypeStruct((B, DP), jnp.float32),
    mesh=_mesh,
    scratch_types=[
        pltpu.VMEM((L, D), jnp.float32),
        pltpu.VMEM((L, D), jnp.float32),
        pltpu.VMEM((L, DP), jnp.float32),
        pltpu.VMEM((L, DP), jnp.float32),
        pltpu.SemaphoreType.DMA,
        pltpu.SemaphoreType.DMA,
    ],
    compiler_params=pltpu.CompilerParams(
        needs_layout_passes=False, use_tc_tiling_on_sc=True),
)
def _sc_histogram(inputs_hbm, c_hbm, blk0, blk1, h0, h1, sem0, sem1):
    wid = lax.axis_index("s") * NC + lax.axis_index("c")
    lanes = lax.iota(jnp.int32, L)
    ones = jnp.ones((L,), jnp.float32)
    zeros16 = jnp.zeros((L,), jnp.float32)
    row0 = wid * ROWS_PER_W
    row1 = row0 + L

    cp0 = pltpu.async_copy(inputs_hbm.at[pl.ds(row0, L)], blk0, sem0)
    cp1 = pltpu.async_copy(inputs_hbm.at[pl.ds(row1, L)], blk1, sem1)

    @plsc.parallel_loop(0, DP // L, step=1)
    def _zero(i):
        off = i * L
        for l in range(L):
            h0[l, pl.ds(off, L)] = zeros16
            h1[l, pl.ds(off, L)] = zeros16

    cp0.wait()
    cp1.wait()

    @plsc.parallel_loop(0, D, step=1, unroll=8,
                        carry=lanes)
    def _scatter(d, dvec):
        iv0 = plsc.load_gather(blk0, [lanes, dvec])
        iv1 = plsc.load_gather(blk1, [lanes, dvec])
        plsc.addupdate_scatter(h0, [lanes, iv0.astype(jnp.int32)], ones)
        plsc.addupdate_scatter(h1, [lanes, iv1.astype(jnp.int32)], ones)
        nd = dvec + 1
        return jnp.where(nd >= D, nd - D, nd)

    wb0 = pltpu.async_copy(h0, c_hbm.at[pl.ds(row0, L)], sem0)
    wb1 = pltpu.async_copy(h1, c_hbm.at[pl.ds(row1, L)], sem1)
    wb0.wait()
    wb1.wait()


BLK = 512


def _tc_logits0(x_ref, w_ref, o_ref):
    o_ref[...] = jnp.dot(x_ref[...], w_ref[...],
                         preferred_element_type=jnp.float32)


def _tc_fm(c_ref, vp_ref, l0_ref, o_ref):
    vp = vp_ref[...]
    r = jnp.sum(vp * vp, axis=1, keepdims=True)
    c = c_ref[...]
    s = jnp.dot(c, vp, preferred_element_type=jnp.float32)
    t1 = jnp.dot(c, r, preferred_element_type=jnp.float32)
    res = l0_ref[...] + 0.5 * (
        jnp.sum(s * s, axis=1, keepdims=True) - t1)
    o_ref[...] = jnp.sum(res, axis=1)


@jax.jit
def kernel(inputs, w, v):
    l0 = pl.pallas_call(
        _tc_logits0,
        grid=(B // BLK,),
        in_specs=[
            pl.BlockSpec((BLK, D), lambda i: (i, 0)),
            pl.BlockSpec((D, 1), lambda i: (0, 0)),
        ],
        out_specs=pl.BlockSpec((BLK, 1), lambda i: (i, 0)),
        out_shape=jax.ShapeDtypeStruct((B, 1), jnp.float32),
    )(inputs, w)
    c = _sc_histogram(inputs)
    v_pad = jnp.pad(v, ((0, DP - D), (0, 0)))
    out = pl.pallas_call(
        _tc_fm,
        grid=(B // BLK,),
        in_specs=[
            pl.BlockSpec((BLK, DP), lambda i: (i, 0)),
            pl.BlockSpec((DP, K), lambda i: (0, 0)),
            pl.BlockSpec((BLK, 1), lambda i: (i, 0)),
        ],
        out_specs=pl.BlockSpec((BLK,), lambda i: (i,)),
        out_shape=jax.ShapeDtypeStruct((B,), jnp.float32),
    )(c, v_pad, l0)
    return out
